# baseline (speedup 1.0000x reference)
import jax
import jax.numpy as jnp
from jax.experimental import pallas as pl
from jax.experimental.pallas import tpu as pltpu

_L, _B, _H, _N = 2, 4, 12, 577
_G = _B * _H
_K = int(_N * 0.5)
_NP = _N - 1


def _scores_body(rgb0_ref, tir0_ref, vrgb_ref, vtir_ref, srgb_ref, stir_ref):
    dn = (((1,), (0,)), ((), ()))
    s_rgb = jax.lax.dot_general(vrgb_ref[0, 0, 0:1, :], rgb0_ref[0, 0], dn,
                                preferred_element_type=jnp.float32)
    s_tir = jax.lax.dot_general(vtir_ref[0, 0, 0:1, :], tir0_ref[0, 0], dn,
                                preferred_element_type=jnp.float32)
    srgb_ref[0] = s_rgb[:, 1:]
    stir_ref[0] = s_tir[:, 1:]


def _mask_body(srgb_ref, stir_ref, out_ref):
    s = jnp.concatenate([srgb_ref[...], stir_ref[...]], axis=0)
    si = jax.lax.bitcast_convert_type(s, jnp.int32)
    lo = jnp.zeros((2 * _G, 1), jnp.int32)
    hi = jnp.full((2 * _G, 1), 0x7F800000, jnp.int32)

    def bisect(_, carry):
        lo, hi = carry
        mid = lo + (hi - lo) // 2
        cnt = jnp.sum((si >= mid).astype(jnp.int32), axis=1, keepdims=True)
        ge = cnt >= _K
        return jnp.where(ge, mid, lo), jnp.where(ge, hi, mid)

    lo, _ = jax.lax.fori_loop(0, 31, bisect, (lo, hi))
    m = (si >= lo).astype(jnp.float32)
    row_b = (jax.lax.broadcasted_iota(jnp.int32, (_B, 2 * _G), 1) // _H) % _B
    sel = (row_b == jax.lax.broadcasted_iota(jnp.int32, (_B, 2 * _G), 0))
    agg = jax.lax.dot_general(sel.astype(jnp.float32), m,
                              (((1,), (0,)), ((), ())),
                              preferred_element_type=jnp.float32)
    union = (agg > 0.0).astype(jnp.float32)
    f = jnp.concatenate([union, jnp.zeros((_B, 1), jnp.float32)], axis=1)
    out_ref[0] = 1.0 - f
    out_ref[1] = f


def kernel(RGB_attn, TIR_attn):
    rgb = RGB_attn.reshape(_L, _G, _N, _N)
    tir = TIR_attn.reshape(_L, _G, _N, _N)

    srgb, stir = pl.pallas_call(
        _scores_body,
        grid=(_G,),
        in_specs=[
            pl.BlockSpec((1, 1, _N, _N), lambda i: (0, i, 0, 0)),
            pl.BlockSpec((1, 1, _N, _N), lambda i: (0, i, 0, 0)),
            pl.BlockSpec((1, 1, 8, _N), lambda i: (1, i, 0, 0)),
            pl.BlockSpec((1, 1, 8, _N), lambda i: (1, i, 0, 0)),
        ],
        out_specs=[
            pl.BlockSpec((1, 1, _NP), lambda i: (i, 0, 0)),
            pl.BlockSpec((1, 1, _NP), lambda i: (i, 0, 0)),
        ],
        out_shape=[
            jax.ShapeDtypeStruct((_G, 1, _NP), jnp.float32),
            jax.ShapeDtypeStruct((_G, 1, _NP), jnp.float32),
        ],
        compiler_params=pltpu.CompilerParams(
            dimension_semantics=("arbitrary",),
        ),
    )(rgb, tir, rgb, tir)

    out = pl.pallas_call(
        _mask_body,
        out_shape=jax.ShapeDtypeStruct((2, _B, _N), jnp.float32),
    )(srgb.reshape(_G, _NP), stir.reshape(_G, _NP))

    return jnp.transpose(out, (1, 2, 0))


# speedup vs baseline: 4.2758x; 4.2758x over previous
import jax
import jax.numpy as jnp
from jax.experimental import pallas as pl
from jax.experimental.pallas import tpu as pltpu

_L, _B, _H, _N = 2, 4, 12, 577
_G = _B * _H
_K = int(_N * 0.5)
_NP = _N - 1


def _scores_body(rgb0_ref, tir0_ref, vrgb_ref, vtir_ref, srgb_ref, stir_ref):
    dn = (((1,), (0,)), ((), ()))
    s_rgb = jax.lax.dot_general(vrgb_ref[0, 0, 0, 0:1, :], rgb0_ref[0, 0, 0],
                                dn, preferred_element_type=jnp.float32)
    s_tir = jax.lax.dot_general(vtir_ref[0, 0, 0, 0:1, :], tir0_ref[0, 0, 0],
                                dn, preferred_element_type=jnp.float32)
    srgb_ref[0] = s_rgb[:, 1:]
    stir_ref[0] = s_tir[:, 1:]


def _mask_body(srgb_ref, stir_ref, out_ref):
    s = jnp.concatenate([srgb_ref[...], stir_ref[...]], axis=0)
    si = jax.lax.bitcast_convert_type(s, jnp.int32)
    lo = jnp.zeros((2 * _G, 1), jnp.int32)
    hi = jnp.full((2 * _G, 1), 0x7F800000, jnp.int32)

    def bisect(_, carry):
        lo, hi = carry
        mid = lo + (hi - lo) // 2
        cnt = jnp.sum((si >= mid).astype(jnp.int32), axis=1, keepdims=True)
        ge = cnt >= _K
        return jnp.where(ge, mid, lo), jnp.where(ge, hi, mid)

    lo, _ = jax.lax.fori_loop(0, 31, bisect, (lo, hi))
    m = (si >= lo).astype(jnp.float32)
    row_b = (jax.lax.broadcasted_iota(jnp.int32, (_B, 2 * _G), 1) // _H) % _B
    sel = (row_b == jax.lax.broadcasted_iota(jnp.int32, (_B, 2 * _G), 0))
    agg = jax.lax.dot_general(sel.astype(jnp.float32), m,
                              (((1,), (0,)), ((), ())),
                              preferred_element_type=jnp.float32)
    union = (agg > 0.0).astype(jnp.float32)
    f = jnp.concatenate([union, jnp.zeros((_B, 1), jnp.float32)], axis=1)
    out_ref[0] = 1.0 - f
    out_ref[1] = f


def kernel(RGB_attn, TIR_attn):
    srgb, stir = pl.pallas_call(
        _scores_body,
        grid=(_G,),
        in_specs=[
            pl.BlockSpec((1, 1, 1, _N, _N), lambda i: (0, i // _H, i % _H, 0, 0)),
            pl.BlockSpec((1, 1, 1, _N, _N), lambda i: (0, i // _H, i % _H, 0, 0)),
            pl.BlockSpec((1, 1, 1, 8, _N), lambda i: (1, i // _H, i % _H, 0, 0)),
            pl.BlockSpec((1, 1, 1, 8, _N), lambda i: (1, i // _H, i % _H, 0, 0)),
        ],
        out_specs=[
            pl.BlockSpec((1, 1, _NP), lambda i: (i, 0, 0)),
            pl.BlockSpec((1, 1, _NP), lambda i: (i, 0, 0)),
        ],
        out_shape=[
            jax.ShapeDtypeStruct((_G, 1, _NP), jnp.float32),
            jax.ShapeDtypeStruct((_G, 1, _NP), jnp.float32),
        ],
        compiler_params=pltpu.CompilerParams(
            dimension_semantics=("arbitrary",),
        ),
    )(RGB_attn, TIR_attn, RGB_attn, TIR_attn)

    out = pl.pallas_call(
        _mask_body,
        out_shape=jax.ShapeDtypeStruct((2, _B, _N), jnp.float32),
    )(srgb.reshape(_G, _NP), stir.reshape(_G, _NP))

    return jnp.transpose(out, (1, 2, 0))
